# baseline (speedup 1.0000x reference)
import jax, jax.numpy as jnp
from jax.experimental import pallas as pl

def _relu_body(x_ref, o_ref):
    o_ref[...] = jnp.maximum(x_ref[...], 0.0)

def kernel(feats, edge_src, edge_dst, W, gamma, beta):
    out = jnp.zeros((feats.shape[0], W.shape[-1]), dtype=feats.dtype)
    for k in range(W.shape[0]):
        gathered = jnp.take(feats, edge_src[k], axis=0)
        msg = gathered @ W[k]
        out = out.at[edge_dst[k]].add(msg)
    mean = jnp.mean(out, axis=0)
    var = jnp.var(out, axis=0)
    y = (out - mean) / jnp.sqrt(var + 1e-5) * gamma + beta
    return pl.pallas_call(_relu_body,
        grid=(25,),
        in_specs=[pl.BlockSpec((2000, 128), lambda i: (i, 0))],
        out_specs=pl.BlockSpec((2000, 128), lambda i: (i, 0)),
        out_shape=jax.ShapeDtypeStruct((50000, 128), jnp.float32))(y)


# speedup vs baseline: 1.0487x; 1.0487x over previous
import jax
import jax.numpy as jnp
from jax import lax
from jax.experimental import pallas as pl
from jax.experimental.pallas import tpu as pltpu
from jax.experimental.pallas import tpu_sc as plsc

N = 50000
C = 128
K = 27
E = 20000
BN_EPS = 1e-5

NC, NS, L = 2, 16, 16
NW = NC * NS

E_PAD = 20480
KE_PAD = K * E_PAD
IDX_ROWS = KE_PAD // 128
G_CHUNKS = KE_PAD // NW // 128
S_CHUNKS = IDX_ROWS // 32

N_ACC = 51200
ACC_ROWS = N_ACC // 128
DUMMY_DST = N

MM_TILE = 2048
RED_TILE = 2048


def _gather_body(feats_hbm, src_hbm, g_hbm, idx_v, buf_v, sem):
    c = lax.axis_index("c")
    s = lax.axis_index("s")
    wid = s * NC + c
    row0 = wid * G_CHUNKS
    pltpu.sync_copy(src_hbm.at[wid], idx_v)

    def step(j, carry):
        pltpu.async_copy(feats_hbm.at[idx_v.at[j]], buf_v, sem).wait()
        off = pl.multiple_of((row0 + j) * 128, 128)
        pltpu.sync_copy(buf_v, g_hbm.at[pl.ds(off, 128)])
        return carry

    lax.fori_loop(0, G_CHUNKS, step, 0)


def _sc_gather(feats, src3):
    mesh = plsc.VectorSubcoreMesh(core_axis_name="c", subcore_axis_name="s")
    return pl.kernel(
        _gather_body,
        out_type=jax.ShapeDtypeStruct((KE_PAD, C), jnp.float32),
        mesh=mesh,
        scratch_types=[
            pltpu.VMEM((G_CHUNKS, 128), jnp.int32),
            pltpu.VMEM((128, C), jnp.float32),
            pltpu.SemaphoreType.DMA,
        ],
    )(feats, src3)


def _mm_body(g_ref, w_ref, o_ref):
    o_ref[...] = lax.dot_general(
        w_ref[0], g_ref[...],
        dimension_numbers=(((0,), (1,)), ((), ())),
        preferred_element_type=jnp.float32)


def _tc_matmul_t(g, w):
    return pl.pallas_call(
        _mm_body,
        grid=(KE_PAD // MM_TILE,),
        in_specs=[
            pl.BlockSpec((MM_TILE, C), lambda i: (i, 0)),
            pl.BlockSpec((1, C, C), lambda i: (i // (E_PAD // MM_TILE), 0, 0)),
        ],
        out_specs=pl.BlockSpec((C, MM_TILE), lambda i: (0, i)),
        out_shape=jax.ShapeDtypeStruct((C, KE_PAD), jnp.float32),
    )(g, w)


def _scatter_body(mt_hbm, dst_hbm, ot_hbm, dstv, m0v, m1v, acc0, acc1):
    c = lax.axis_index("c")
    s = lax.axis_index("s")
    wid = s * NC + c

    for p in range(2):
        ch0 = (p * NW + wid) * 2

        def zero_row(r, carry):
            z = jnp.zeros((L,), jnp.float32)
            acc0[pl.ds(r * L, L)] = z
            acc1[pl.ds(r * L, L)] = z
            return carry

        lax.fori_loop(0, N_ACC // L, zero_row, 0)

        def step(j, carry):
            off = pl.multiple_of(j * 32, 32)
            pltpu.sync_copy(dst_hbm.at[pl.ds(off, 32)], dstv)
            pltpu.sync_copy(mt_hbm.at[ch0, pl.ds(off, 32)], m0v)
            pltpu.sync_copy(mt_hbm.at[ch0 + 1, pl.ds(off, 32)], m1v)

            def inner(r, carry2):
                for v in range(8):
                    d = dstv[r, pl.ds(v * L, L)]
                    plsc.addupdate_scatter(acc0, [d],
                                           m0v[r, pl.ds(v * L, L)])
                    plsc.addupdate_scatter(acc1, [d],
                                           m1v[r, pl.ds(v * L, L)])
                return carry2

            lax.fori_loop(0, 32, inner, 0)
            return carry

        lax.fori_loop(0, S_CHUNKS, step, 0)
        pltpu.sync_copy(acc0, ot_hbm.at[pl.ds(ch0 * N_ACC, N_ACC)])
        pltpu.sync_copy(acc1, ot_hbm.at[pl.ds((ch0 + 1) * N_ACC, N_ACC)])


def _sc_scatter(mt3, dst2):
    mesh = plsc.VectorSubcoreMesh(core_axis_name="c", subcore_axis_name="s")
    return pl.kernel(
        _scatter_body,
        out_type=jax.ShapeDtypeStruct((C * N_ACC,), jnp.float32),
        mesh=mesh,
        scratch_types=[
            pltpu.VMEM((32, 128), jnp.int32),
            pltpu.VMEM((32, 128), jnp.float32),
            pltpu.VMEM((32, 128), jnp.float32),
            pltpu.VMEM((N_ACC,), jnp.float32),
            pltpu.VMEM((N_ACC,), jnp.float32),
        ],
        compiler_params=pltpu.CompilerParams(needs_layout_passes=False),
    )(mt3, dst2)


def _red_body(x_ref, o_ref):
    i = pl.program_id(0)
    x = x_ref[...]
    col = lax.broadcasted_iota(jnp.int32, (C, RED_TILE), 1) + i * RED_TILE
    x = jnp.where(col < N, x, 0.0)
    ps = jnp.sum(x, axis=1, keepdims=True)
    pss = jnp.sum(x * x, axis=1, keepdims=True)

    @pl.when(i == 0)
    def _():
        o_ref[...] = jnp.zeros_like(o_ref)

    o_ref[:, 0:1] = o_ref[:, 0:1] + ps
    o_ref[:, 1:2] = o_ref[:, 1:2] + pss


def _tc_reduce(ot2):
    return pl.pallas_call(
        _red_body,
        grid=(N_ACC // RED_TILE,),
        in_specs=[pl.BlockSpec((C, RED_TILE), lambda i: (0, i))],
        out_specs=pl.BlockSpec((C, 128), lambda i: (0, 0)),
        out_shape=jax.ShapeDtypeStruct((C, 128), jnp.float32),
    )(ot2)


def _apply_body(x_ref, st_ref, gb_ref, o_ref):
    x = x_ref[...]
    inv_n = 1.0 / N
    mean = st_ref[:, 0:1] * inv_n
    var = st_ref[:, 1:2] * inv_n - mean * mean
    scale = gb_ref[:, 0:1] * lax.rsqrt(var + BN_EPS)
    shift = gb_ref[:, 1:2] - mean * scale
    y = jnp.maximum(x * scale + shift, 0.0)
    r = lax.broadcasted_iota(jnp.int32, (C, C), 0)
    cc = lax.broadcasted_iota(jnp.int32, (C, C), 1)
    eye = jnp.where(r == cc, 1.0, 0.0).astype(jnp.float32)
    o_ref[...] = lax.dot_general(
        y, eye, dimension_numbers=(((0,), (0,)), ((), ())),
        preferred_element_type=jnp.float32)


def _tc_apply(ot2, stats, gb):
    return pl.pallas_call(
        _apply_body,
        grid=(N_ACC // RED_TILE,),
        in_specs=[
            pl.BlockSpec((C, RED_TILE), lambda i: (0, i)),
            pl.BlockSpec((C, 128), lambda i: (0, 0)),
            pl.BlockSpec((C, 128), lambda i: (0, 0)),
        ],
        out_specs=pl.BlockSpec((RED_TILE, C), lambda i: (i, 0)),
        out_shape=jax.ShapeDtypeStruct((N, C), jnp.float32),
    )(ot2, stats, gb)


@jax.jit
def kernel(feats, edge_src, edge_dst, W, gamma, beta):
    src = jnp.pad(edge_src.astype(jnp.int32), ((0, 0), (0, E_PAD - E)))
    dst = jnp.pad(edge_dst.astype(jnp.int32), ((0, 0), (0, E_PAD - E)),
                  constant_values=DUMMY_DST)
    src3 = src.reshape(NW, G_CHUNKS, 128)
    dst2 = dst.reshape(IDX_ROWS, 128)

    g = _sc_gather(feats, src3)
    mt = _tc_matmul_t(g, W)
    ot = _sc_scatter(mt.reshape(C, IDX_ROWS, 128), dst2)
    ot2 = ot.reshape(C, N_ACC)
    stats = _tc_reduce(ot2)
    gb = jnp.zeros((C, 128), jnp.float32)
    gb = gb.at[:, 0].set(gamma).at[:, 1].set(beta)
    return _tc_apply(ot2, stats, gb)


# speedup vs baseline: 1.3294x; 1.2677x over previous
import jax
import jax.numpy as jnp
from jax import lax
from jax.experimental import pallas as pl
from jax.experimental.pallas import tpu as pltpu
from jax.experimental.pallas import tpu_sc as plsc

N = 50000
C = 128
K = 27
E = 20000
BN_EPS = 1e-5

NC, NS, L = 2, 16, 16
NW = NC * NS

E_PAD = 20480
KE_PAD = K * E_PAD
IDX_ROWS = KE_PAD // 128
EW = KE_PAD // NW

GR = 128
G_CHUNKS = EW // GR
SR = 24
S_CHUNKS = IDX_ROWS // SR

N_ACC = 51200
DUMMY_DST = N

MM_TILE = 2048
RED_TILE = 2048


def _pad_body(s_ref, d_ref, so_ref, do_ref):
    so_ref[:, :E] = s_ref[...]
    so_ref[:, E:] = jnp.zeros((K, E_PAD - E), jnp.int32)
    do_ref[:, :E] = d_ref[...]
    do_ref[:, E:] = jnp.full((K, E_PAD - E), DUMMY_DST, jnp.int32)


def _tc_pad(src, dst):
    return pl.pallas_call(
        _pad_body,
        out_shape=(jax.ShapeDtypeStruct((K, E_PAD), jnp.int32),
                   jax.ShapeDtypeStruct((K, E_PAD), jnp.int32)),
    )(src, dst)


def _gather_body(feats_hbm, src_hbm, g_hbm, idx_v, bufa, bufb,
                 sga, sgb, swa, swb):
    c = lax.axis_index("c")
    s = lax.axis_index("s")
    wid = s * NC + c
    row0 = wid * (EW // 128)
    pltpu.sync_copy(src_hbm.at[wid], idx_v)

    def fire_g(j, buf, sem):
        pltpu.async_copy(feats_hbm.at[idx_v.at[j]], buf, sem)

    def wait_g(buf, sem):
        pltpu.make_async_copy(feats_hbm.at[pl.ds(0, GR)], buf, sem).wait()

    def fire_w(j, buf, sem):
        off = pl.multiple_of((row0 + j) * GR, GR)
        pltpu.async_copy(buf, g_hbm.at[pl.ds(off, GR)], sem)

    def wait_w(buf, sem):
        pltpu.make_async_copy(buf, g_hbm.at[pl.ds(0, GR)], sem).wait()

    fire_g(0, bufa, sga)

    def body(i, carry):
        @pl.when(i > 0)
        def _():
            wait_w(bufb, swb)
        fire_g(2 * i + 1, bufb, sgb)
        wait_g(bufa, sga)
        fire_w(2 * i, bufa, swa)
        wait_w(bufa, swa)
        fire_g(2 * i + 2, bufa, sga)
        wait_g(bufb, sgb)
        fire_w(2 * i + 1, bufb, swb)
        return carry

    lax.fori_loop(0, G_CHUNKS // 2, body, 0)
    wait_w(bufb, swb)
    wait_g(bufa, sga)
    fire_w(G_CHUNKS - 1, bufa, swa)
    wait_w(bufa, swa)


def _sc_gather(feats, src3):
    mesh = plsc.VectorSubcoreMesh(core_axis_name="c", subcore_axis_name="s")
    return pl.kernel(
        _gather_body,
        out_type=jax.ShapeDtypeStruct((KE_PAD, C), jnp.float32),
        mesh=mesh,
        scratch_types=[
            pltpu.VMEM((EW // 128, 128), jnp.int32),
            pltpu.VMEM((GR, C), jnp.float32),
            pltpu.VMEM((GR, C), jnp.float32),
            pltpu.SemaphoreType.DMA,
            pltpu.SemaphoreType.DMA,
            pltpu.SemaphoreType.DMA,
            pltpu.SemaphoreType.DMA,
        ],
        compiler_params=pltpu.CompilerParams(needs_layout_passes=False),
    )(feats, src3)


def _scatter_body(mt_hbm, dst_hbm, ot_hbm, dsta, dstb, ma, mb,
                  acc0, acc1, sa, sb):
    c = lax.axis_index("c")
    s = lax.axis_index("s")
    wid = s * NC + c

    for p in range(2):
        ch0 = (p * NW + wid) * 2

        def zero_row(r, carry):
            z = jnp.zeros((L,), jnp.float32)
            acc0[pl.ds(r * L, L)] = z
            acc1[pl.ds(r * L, L)] = z
            return carry

        lax.fori_loop(0, N_ACC // L, zero_row, 0)

        def fire(j, dbuf, mbuf, sem):
            off = pl.multiple_of(j * SR, SR)
            pltpu.async_copy(dst_hbm.at[pl.ds(off, SR)], dbuf, sem)
            pltpu.async_copy(
                mt_hbm.at[pl.ds(ch0, 2), pl.ds(off, SR)], mbuf, sem)

        def wait(dbuf, mbuf, sem):
            pltpu.make_async_copy(dst_hbm.at[pl.ds(0, SR)], dbuf, sem).wait()
            pltpu.make_async_copy(
                mt_hbm.at[pl.ds(0, 2), pl.ds(0, SR)], mbuf, sem).wait()

        def compute(dbuf, mbuf):
            def inner(r, carry2):
                for v in range(8):
                    d = dbuf[r, pl.ds(v * L, L)]
                    plsc.addupdate_scatter(acc0, [d],
                                           mbuf[0, r, pl.ds(v * L, L)])
                    plsc.addupdate_scatter(acc1, [d],
                                           mbuf[1, r, pl.ds(v * L, L)])
                return carry2
            lax.fori_loop(0, SR, inner, 0)

        fire(0, dsta, ma, sa)

        def body(i, carry):
            fire(2 * i + 1, dstb, mb, sb)
            wait(dsta, ma, sa)
            compute(dsta, ma)

            @pl.when(i < S_CHUNKS // 2 - 1)
            def _():
                fire(2 * i + 2, dsta, ma, sa)
            wait(dstb, mb, sb)
            compute(dstb, mb)
            return carry

        lax.fori_loop(0, S_CHUNKS // 2, body, 0)
        pltpu.sync_copy(acc0, ot_hbm.at[pl.ds(ch0 * N_ACC, N_ACC)])
        pltpu.sync_copy(acc1, ot_hbm.at[pl.ds((ch0 + 1) * N_ACC, N_ACC)])


def _sc_scatter(mt3, dst2):
    mesh = plsc.VectorSubcoreMesh(core_axis_name="c", subcore_axis_name="s")
    return pl.kernel(
        _scatter_body,
        out_type=jax.ShapeDtypeStruct((C * N_ACC,), jnp.float32),
        mesh=mesh,
        scratch_types=[
            pltpu.VMEM((SR, 128), jnp.int32),
            pltpu.VMEM((SR, 128), jnp.int32),
            pltpu.VMEM((2, SR, 128), jnp.float32),
            pltpu.VMEM((2, SR, 128), jnp.float32),
            pltpu.VMEM((N_ACC,), jnp.float32),
            pltpu.VMEM((N_ACC,), jnp.float32),
            pltpu.SemaphoreType.DMA,
            pltpu.SemaphoreType.DMA,
        ],
        compiler_params=pltpu.CompilerParams(needs_layout_passes=False),
    )(mt3, dst2)


def _mm_body(g_ref, w_ref, o_ref):
    o_ref[...] = lax.dot_general(
        w_ref[0], g_ref[...],
        dimension_numbers=(((0,), (1,)), ((), ())),
        preferred_element_type=jnp.float32)


def _tc_matmul_t(g, w):
    return pl.pallas_call(
        _mm_body,
        grid=(KE_PAD // MM_TILE,),
        in_specs=[
            pl.BlockSpec((MM_TILE, C), lambda i: (i, 0)),
            pl.BlockSpec((1, C, C), lambda i: (i // (E_PAD // MM_TILE), 0, 0)),
        ],
        out_specs=pl.BlockSpec((C, MM_TILE), lambda i: (0, i)),
        out_shape=jax.ShapeDtypeStruct((C, KE_PAD), jnp.float32),
    )(g, w)


def _red_body(x_ref, o_ref):
    i = pl.program_id(0)
    x = x_ref[...]
    col = lax.broadcasted_iota(jnp.int32, (C, RED_TILE), 1) + i * RED_TILE
    x = jnp.where(col < N, x, 0.0)
    ps = jnp.sum(x, axis=1, keepdims=True)
    pss = jnp.sum(x * x, axis=1, keepdims=True)

    @pl.when(i == 0)
    def _():
        o_ref[...] = jnp.zeros_like(o_ref)

    o_ref[:, 0:1] = o_ref[:, 0:1] + ps
    o_ref[:, 1:2] = o_ref[:, 1:2] + pss


def _tc_reduce(ot2):
    return pl.pallas_call(
        _red_body,
        grid=(N_ACC // RED_TILE,),
        in_specs=[pl.BlockSpec((C, RED_TILE), lambda i: (0, i))],
        out_specs=pl.BlockSpec((C, 128), lambda i: (0, 0)),
        out_shape=jax.ShapeDtypeStruct((C, 128), jnp.float32),
    )(ot2)


def _apply_body(x_ref, st_ref, gb_ref, o_ref):
    x = x_ref[...]
    inv_n = 1.0 / N
    mean = st_ref[:, 0:1] * inv_n
    var = st_ref[:, 1:2] * inv_n - mean * mean
    scale = gb_ref[:, 0:1] * lax.rsqrt(var + BN_EPS)
    shift = gb_ref[:, 1:2] - mean * scale
    y = jnp.maximum(x * scale + shift, 0.0)
    r = lax.broadcasted_iota(jnp.int32, (C, C), 0)
    cc = lax.broadcasted_iota(jnp.int32, (C, C), 1)
    eye = jnp.where(r == cc, 1.0, 0.0).astype(jnp.float32)
    o_ref[...] = lax.dot_general(
        y, eye, dimension_numbers=(((0,), (0,)), ((), ())),
        preferred_element_type=jnp.float32)


def _tc_apply(ot2, stats, gb):
    return pl.pallas_call(
        _apply_body,
        grid=(N_ACC // RED_TILE,),
        in_specs=[
            pl.BlockSpec((C, RED_TILE), lambda i: (0, i)),
            pl.BlockSpec((C, 128), lambda i: (0, 0)),
            pl.BlockSpec((C, 128), lambda i: (0, 0)),
        ],
        out_specs=pl.BlockSpec((RED_TILE, C), lambda i: (i, 0)),
        out_shape=jax.ShapeDtypeStruct((N, C), jnp.float32),
    )(ot2, stats, gb)


@jax.jit
def kernel(feats, edge_src, edge_dst, W, gamma, beta):
    src_p, dst_p = _tc_pad(edge_src.astype(jnp.int32),
                           edge_dst.astype(jnp.int32))
    src3 = src_p.reshape(NW, EW // 128, 128)
    dst2 = dst_p.reshape(IDX_ROWS, 128)

    g = _sc_gather(feats, src3)
    mt = _tc_matmul_t(g, W)
    ot = _sc_scatter(mt.reshape(C, IDX_ROWS, 128), dst2)
    ot2 = ot.reshape(C, N_ACC)
    stats = _tc_reduce(ot2)
    gb = jnp.zeros((C, 128), jnp.float32)
    gb = gb.at[:, 0].set(gamma).at[:, 1].set(beta)
    return _tc_apply(ot2, stats, gb)


# speedup vs baseline: 1.3349x; 1.0041x over previous
import jax
import jax.numpy as jnp
from jax import lax
from jax.experimental import pallas as pl
from jax.experimental.pallas import tpu as pltpu
from jax.experimental.pallas import tpu_sc as plsc

N = 50000
C = 128
K = 27
E = 20000
BN_EPS = 1e-5

NC, NS, L = 2, 16, 16
NW = NC * NS

E_PAD = 20480
KE_PAD = K * E_PAD
IDX_ROWS = KE_PAD // 128
EW = KE_PAD // NW

GR = 128
G_CHUNKS = EW // GR
SR = 24
S_CHUNKS = IDX_ROWS // SR

N_ACC = 51200
DUMMY_DST = N

MM_TILE = 2048
RED_TILE = 2048


def _pad_body(s_ref, d_ref, so_ref, do_ref):
    so_ref[:, :E] = s_ref[...]
    so_ref[:, E:] = jnp.zeros((K, E_PAD - E), jnp.int32)
    do_ref[:, :E] = d_ref[...]
    do_ref[:, E:] = jnp.full((K, E_PAD - E), DUMMY_DST, jnp.int32)


def _tc_pad(src, dst):
    return pl.pallas_call(
        _pad_body,
        out_shape=(jax.ShapeDtypeStruct((K, E_PAD), jnp.int32),
                   jax.ShapeDtypeStruct((K, E_PAD), jnp.int32)),
    )(src, dst)


NBUF = 5


def _gather_body(feats_hbm, src_hbm, g_hbm, idx_v, *bufs_and_sems):
    bufs = bufs_and_sems[:NBUF]
    gsems = bufs_and_sems[NBUF:2 * NBUF]
    wsems = bufs_and_sems[2 * NBUF:3 * NBUF]
    c = lax.axis_index("c")
    s = lax.axis_index("s")
    wid = s * NC + c
    row0 = wid * (EW // 128)
    pltpu.sync_copy(src_hbm.at[wid], idx_v)

    def fire_g(j, b):
        pltpu.async_copy(feats_hbm.at[idx_v.at[j]], bufs[b], gsems[b])

    def wait_g(b):
        pltpu.make_async_copy(feats_hbm.at[pl.ds(0, GR)],
                              bufs[b], gsems[b]).wait()

    def fire_w(j, b):
        off = pl.multiple_of((row0 + j) * GR, GR)
        pltpu.async_copy(bufs[b], g_hbm.at[pl.ds(off, GR)], wsems[b])

    def wait_w(b):
        pltpu.make_async_copy(bufs[b], g_hbm.at[pl.ds(0, GR)],
                              wsems[b]).wait()

    def body(i, carry):
        for b in range(NBUF):
            @pl.when(i > 0)
            def _():
                wait_w(b)
            fire_g(i * NBUF + b, b)
        for b in range(NBUF):
            wait_g(b)
            fire_w(i * NBUF + b, b)
        return carry

    lax.fori_loop(0, G_CHUNKS // NBUF, body, 0)
    for b in range(NBUF):
        wait_w(b)


def _sc_gather(feats, src3):
    mesh = plsc.VectorSubcoreMesh(core_axis_name="c", subcore_axis_name="s")
    return pl.kernel(
        _gather_body,
        out_type=jax.ShapeDtypeStruct((KE_PAD, C), jnp.float32),
        mesh=mesh,
        scratch_types=[pltpu.VMEM((EW // 128, 128), jnp.int32)]
        + [pltpu.VMEM((GR, C), jnp.float32)] * NBUF
        + [pltpu.SemaphoreType.DMA] * (2 * NBUF),
        compiler_params=pltpu.CompilerParams(needs_layout_passes=False),
    )(feats, src3)


SBUF = 3


def _scatter_body(mt_hbm, dst_hbm, ot_hbm, acc0, acc1, *bufs_and_sems):
    dbufs = bufs_and_sems[:SBUF]
    mbufs = bufs_and_sems[SBUF:2 * SBUF]
    sems = bufs_and_sems[2 * SBUF:3 * SBUF]
    c = lax.axis_index("c")
    s = lax.axis_index("s")
    wid = s * NC + c

    for p in range(2):
        ch0 = (p * NW + wid) * 2

        def zero_row(r, carry):
            z = jnp.zeros((L,), jnp.float32)
            acc0[pl.ds(r * L, L)] = z
            acc1[pl.ds(r * L, L)] = z
            return carry

        lax.fori_loop(0, N_ACC // L, zero_row, 0)

        def fire(j, b):
            off = pl.multiple_of(j * SR, SR)
            pltpu.async_copy(dst_hbm.at[pl.ds(off, SR)], dbufs[b], sems[b])
            pltpu.async_copy(
                mt_hbm.at[pl.ds(ch0, 2), pl.ds(off, SR)], mbufs[b], sems[b])

        def wait(b):
            pltpu.make_async_copy(dst_hbm.at[pl.ds(0, SR)],
                                  dbufs[b], sems[b]).wait()
            pltpu.make_async_copy(mt_hbm.at[pl.ds(0, 2), pl.ds(0, SR)],
                                  mbufs[b], sems[b]).wait()

        def compute(b):
            dbuf, mbuf = dbufs[b], mbufs[b]

            def inner(r, carry2):
                for v in range(8):
                    d = dbuf[r, pl.ds(v * L, L)]
                    plsc.addupdate_scatter(acc0, [d],
                                           mbuf[0, r, pl.ds(v * L, L)])
                    plsc.addupdate_scatter(acc1, [d],
                                           mbuf[1, r, pl.ds(v * L, L)])
                return carry2
            lax.fori_loop(0, SR, inner, 0)

        for b in range(SBUF):
            fire(b, b)

        def body(i, carry):
            for b in range(SBUF):
                wait(b)
                compute(b)

                @pl.when(i < S_CHUNKS // SBUF - 1)
                def _():
                    fire((i + 1) * SBUF + b, b)
            return carry

        lax.fori_loop(0, S_CHUNKS // SBUF, body, 0)
        pltpu.sync_copy(acc0, ot_hbm.at[pl.ds(ch0 * N_ACC, N_ACC)])
        pltpu.sync_copy(acc1, ot_hbm.at[pl.ds((ch0 + 1) * N_ACC, N_ACC)])


def _sc_scatter(mt3, dst2):
    mesh = plsc.VectorSubcoreMesh(core_axis_name="c", subcore_axis_name="s")
    return pl.kernel(
        _scatter_body,
        out_type=jax.ShapeDtypeStruct((C * N_ACC,), jnp.float32),
        mesh=mesh,
        scratch_types=[
            pltpu.VMEM((N_ACC,), jnp.float32),
            pltpu.VMEM((N_ACC,), jnp.float32),
        ]
        + [pltpu.VMEM((SR, 128), jnp.int32)] * SBUF
        + [pltpu.VMEM((2, SR, 128), jnp.float32)] * SBUF
        + [pltpu.SemaphoreType.DMA] * SBUF,
        compiler_params=pltpu.CompilerParams(needs_layout_passes=False),
    )(mt3, dst2)


def _mm_body(g_ref, w_ref, o_ref):
    o_ref[...] = lax.dot_general(
        w_ref[0], g_ref[...],
        dimension_numbers=(((0,), (1,)), ((), ())),
        preferred_element_type=jnp.float32)


def _tc_matmul_t(g, w):
    return pl.pallas_call(
        _mm_body,
        grid=(KE_PAD // MM_TILE,),
        in_specs=[
            pl.BlockSpec((MM_TILE, C), lambda i: (i, 0)),
            pl.BlockSpec((1, C, C), lambda i: (i // (E_PAD // MM_TILE), 0, 0)),
        ],
        out_specs=pl.BlockSpec((C, MM_TILE), lambda i: (0, i)),
        out_shape=jax.ShapeDtypeStruct((C, KE_PAD), jnp.float32),
    )(g, w)


def _red_body(x_ref, o_ref):
    i = pl.program_id(0)
    x = x_ref[...]
    col = lax.broadcasted_iota(jnp.int32, (C, RED_TILE), 1) + i * RED_TILE
    x = jnp.where(col < N, x, 0.0)
    ps = jnp.sum(x, axis=1, keepdims=True)
    pss = jnp.sum(x * x, axis=1, keepdims=True)

    @pl.when(i == 0)
    def _():
        o_ref[...] = jnp.zeros_like(o_ref)

    o_ref[:, 0:1] = o_ref[:, 0:1] + ps
    o_ref[:, 1:2] = o_ref[:, 1:2] + pss


def _tc_reduce(ot2):
    return pl.pallas_call(
        _red_body,
        grid=(N_ACC // RED_TILE,),
        in_specs=[pl.BlockSpec((C, RED_TILE), lambda i: (0, i))],
        out_specs=pl.BlockSpec((C, 128), lambda i: (0, 0)),
        out_shape=jax.ShapeDtypeStruct((C, 128), jnp.float32),
    )(ot2)


def _apply_body(x_ref, st_ref, gb_ref, o_ref):
    x = x_ref[...]
    inv_n = 1.0 / N
    mean = st_ref[:, 0:1] * inv_n
    var = st_ref[:, 1:2] * inv_n - mean * mean
    scale = gb_ref[:, 0:1] * lax.rsqrt(var + BN_EPS)
    shift = gb_ref[:, 1:2] - mean * scale
    y = jnp.maximum(x * scale + shift, 0.0)
    r = lax.broadcasted_iota(jnp.int32, (C, C), 0)
    cc = lax.broadcasted_iota(jnp.int32, (C, C), 1)
    eye = jnp.where(r == cc, 1.0, 0.0).astype(jnp.float32)
    o_ref[...] = lax.dot_general(
        y, eye, dimension_numbers=(((0,), (0,)), ((), ())),
        preferred_element_type=jnp.float32)


def _tc_apply(ot2, stats, gb):
    return pl.pallas_call(
        _apply_body,
        grid=(N_ACC // RED_TILE,),
        in_specs=[
            pl.BlockSpec((C, RED_TILE), lambda i: (0, i)),
            pl.BlockSpec((C, 128), lambda i: (0, 0)),
            pl.BlockSpec((C, 128), lambda i: (0, 0)),
        ],
        out_specs=pl.BlockSpec((RED_TILE, C), lambda i: (i, 0)),
        out_shape=jax.ShapeDtypeStruct((N, C), jnp.float32),
    )(ot2, stats, gb)


@jax.jit
def kernel(feats, edge_src, edge_dst, W, gamma, beta):
    src_p, dst_p = _tc_pad(edge_src.astype(jnp.int32),
                           edge_dst.astype(jnp.int32))
    src3 = src_p.reshape(NW, EW // 128, 128)
    dst2 = dst_p.reshape(IDX_ROWS, 128)

    g = _sc_gather(feats, src3)
    mt = _tc_matmul_t(g, W)
    ot = _sc_scatter(mt.reshape(C, IDX_ROWS, 128), dst2)
    ot2 = ot.reshape(C, N_ACC)
    stats = _tc_reduce(ot2)
    gb = jnp.zeros((C, 128), jnp.float32)
    gb = gb.at[:, 0].set(gamma).at[:, 1].set(beta)
    return _tc_apply(ot2, stats, gb)


# speedup vs baseline: 1.5478x; 1.1595x over previous
import jax
import jax.numpy as jnp
from jax import lax
from jax.experimental import pallas as pl
from jax.experimental.pallas import tpu as pltpu
from jax.experimental.pallas import tpu_sc as plsc

N = 50000
C = 128
K = 27
E = 20000
BN_EPS = 1e-5

NC, NS, L = 2, 16, 16
NW = NC * NS

E_PAD = 20480
KE_PAD = K * E_PAD
IDX_ROWS = KE_PAD // 128
EW = KE_PAD // NW

GR = 128
G_CHUNKS = EW // GR
SR = 24
S_CHUNKS = IDX_ROWS // SR

N_ACC = 51200
DUMMY_DST = N

MM_TILE = 2048
RED_TILE = 2048


def _pad_body(s_ref, d_ref, so_ref, do_ref):
    so_ref[:, :E] = s_ref[...]
    so_ref[:, E:] = jnp.zeros((K, E_PAD - E), jnp.int32)
    do_ref[:, :E] = d_ref[...]
    do_ref[:, E:] = jnp.full((K, E_PAD - E), DUMMY_DST, jnp.int32)


def _tc_pad(src, dst):
    return pl.pallas_call(
        _pad_body,
        out_shape=(jax.ShapeDtypeStruct((K, E_PAD), jnp.int32),
                   jax.ShapeDtypeStruct((K, E_PAD), jnp.int32)),
    )(src, dst)


NBUF = 5


def _gather_body(feats_hbm, src_hbm, g_hbm, idx_v, *bufs_and_sems):
    bufs = bufs_and_sems[:NBUF]
    gsems = bufs_and_sems[NBUF:2 * NBUF]
    wsems = bufs_and_sems[2 * NBUF:3 * NBUF]
    c = lax.axis_index("c")
    s = lax.axis_index("s")
    wid = s * NC + c
    row0 = wid * (EW // 128)
    pltpu.sync_copy(src_hbm.at[wid], idx_v)

    def fire_g(j, b):
        pltpu.async_copy(feats_hbm.at[idx_v.at[j]], bufs[b], gsems[b])

    def wait_g(b):
        pltpu.make_async_copy(feats_hbm.at[pl.ds(0, GR)],
                              bufs[b], gsems[b]).wait()

    def fire_w(j, b):
        off = pl.multiple_of((row0 + j) * GR, GR)
        pltpu.async_copy(bufs[b], g_hbm.at[pl.ds(off, GR)], wsems[b])

    def wait_w(b):
        pltpu.make_async_copy(bufs[b], g_hbm.at[pl.ds(0, GR)],
                              wsems[b]).wait()

    def body(i, carry):
        for b in range(NBUF):
            @pl.when(i > 0)
            def _():
                wait_w(b)
            fire_g(i * NBUF + b, b)
        for b in range(NBUF):
            wait_g(b)
            fire_w(i * NBUF + b, b)
        return carry

    lax.fori_loop(0, G_CHUNKS // NBUF, body, 0)
    for b in range(NBUF):
        wait_w(b)


def _sc_gather(feats, src3):
    mesh = plsc.VectorSubcoreMesh(core_axis_name="c", subcore_axis_name="s")
    return pl.kernel(
        _gather_body,
        out_type=jax.ShapeDtypeStruct((KE_PAD, C), jnp.float32),
        mesh=mesh,
        scratch_types=[pltpu.VMEM((EW // 128, 128), jnp.int32)]
        + [pltpu.VMEM((GR, C), jnp.float32)] * NBUF
        + [pltpu.SemaphoreType.DMA] * (2 * NBUF),
        compiler_params=pltpu.CompilerParams(needs_layout_passes=False),
    )(feats, src3)


SBUF = 3


def _scatter_body(mt_hbm, dst_hbm, ot_hbm, acc0, acc1, *bufs_and_sems):
    dbufs = bufs_and_sems[:SBUF]
    mbufs = bufs_and_sems[SBUF:2 * SBUF]
    sems = bufs_and_sems[2 * SBUF:3 * SBUF]
    c = lax.axis_index("c")
    s = lax.axis_index("s")
    wid = s * NC + c

    for p in range(2):
        rp = p * NW + wid

        def zero_row(r, carry):
            z = jnp.zeros((L,), jnp.float32)
            acc0[pl.ds(r * L, L)] = z
            acc1[pl.ds(r * L, L)] = z
            return carry

        lax.fori_loop(0, N_ACC // L, zero_row, 0)

        def fire(j, b):
            off = pl.multiple_of(j * SR, SR)
            pltpu.async_copy(dst_hbm.at[pl.ds(off, SR)], dbufs[b], sems[b])
            pltpu.async_copy(mt_hbm.at[rp, pl.ds(off, SR)], mbufs[b], sems[b])

        def wait(b):
            pltpu.make_async_copy(dst_hbm.at[pl.ds(0, SR)],
                                  dbufs[b], sems[b]).wait()
            pltpu.make_async_copy(mt_hbm.at[0, pl.ds(0, SR)],
                                  mbufs[b], sems[b]).wait()

        def compute(b):
            dbuf, mbuf = dbufs[b], mbufs[b]

            def inner(r, carry2):
                for v in range(8):
                    d = dbuf[r, pl.ds(v * L, L)]
                    m = mbuf[r, pl.ds(v * L, L)]
                    fe = plsc.bitcast(lax.shift_left(m, 16), jnp.float32)
                    fo = plsc.bitcast(
                        jnp.bitwise_and(m, jnp.int32(-65536)), jnp.float32)
                    plsc.addupdate_scatter(acc0, [d], fe)
                    plsc.addupdate_scatter(acc1, [d], fo)
                return carry2
            lax.fori_loop(0, SR, inner, 0)

        for b in range(SBUF):
            fire(b, b)

        def body(i, carry):
            for b in range(SBUF):
                wait(b)
                compute(b)

                @pl.when(i < S_CHUNKS // SBUF - 1)
                def _():
                    fire((i + 1) * SBUF + b, b)
            return carry

        lax.fori_loop(0, S_CHUNKS // SBUF, body, 0)
        pltpu.sync_copy(acc0, ot_hbm.at[pl.ds((2 * rp) * N_ACC, N_ACC)])
        pltpu.sync_copy(acc1, ot_hbm.at[pl.ds((2 * rp + 1) * N_ACC, N_ACC)])


def _sc_scatter(mt3, dst2):
    mesh = plsc.VectorSubcoreMesh(core_axis_name="c", subcore_axis_name="s")
    return pl.kernel(
        _scatter_body,
        out_type=jax.ShapeDtypeStruct((C * N_ACC,), jnp.float32),
        mesh=mesh,
        scratch_types=[
            pltpu.VMEM((N_ACC,), jnp.float32),
            pltpu.VMEM((N_ACC,), jnp.float32),
        ]
        + [pltpu.VMEM((SR, 128), jnp.int32)] * SBUF
        + [pltpu.VMEM((SR, 128), jnp.int32)] * SBUF
        + [pltpu.SemaphoreType.DMA] * SBUF,
        compiler_params=pltpu.CompilerParams(needs_layout_passes=False),
    )(mt3, dst2)


def _rte_bf16_bits(u):
    lsb = jnp.bitwise_and(lax.shift_right_logical(u, 16), 1)
    return lax.shift_right_logical(u + 0x7FFF + lsb, 16)


def _mm_body(g_ref, we_ref, wo_ref, o_ref):
    g = g_ref[...]
    dn = (((0,), (1,)), ((), ()))
    me = lax.dot_general(we_ref[0], g, dimension_numbers=dn,
                         preferred_element_type=jnp.float32)
    mo = lax.dot_general(wo_ref[0], g, dimension_numbers=dn,
                         preferred_element_type=jnp.float32)
    be = _rte_bf16_bits(lax.bitcast_convert_type(me, jnp.int32))
    bo = _rte_bf16_bits(lax.bitcast_convert_type(mo, jnp.int32))
    o_ref[...] = jnp.bitwise_or(lax.shift_left(bo, 16), be)


def _tc_matmul_t(g, we, wo):
    return pl.pallas_call(
        _mm_body,
        grid=(KE_PAD // MM_TILE,),
        in_specs=[
            pl.BlockSpec((MM_TILE, C), lambda i: (i, 0)),
            pl.BlockSpec((1, C, C // 2),
                         lambda i: (i // (E_PAD // MM_TILE), 0, 0)),
            pl.BlockSpec((1, C, C // 2),
                         lambda i: (i // (E_PAD // MM_TILE), 0, 0)),
        ],
        out_specs=pl.BlockSpec((C // 2, MM_TILE), lambda i: (0, i)),
        out_shape=jax.ShapeDtypeStruct((C // 2, KE_PAD), jnp.int32),
    )(g, we, wo)


def _red_body(x_ref, o_ref):
    i = pl.program_id(0)
    x = x_ref[...]
    col = lax.broadcasted_iota(jnp.int32, (C, RED_TILE), 1) + i * RED_TILE
    x = jnp.where(col < N, x, 0.0)
    ps = jnp.sum(x, axis=1, keepdims=True)
    pss = jnp.sum(x * x, axis=1, keepdims=True)

    @pl.when(i == 0)
    def _():
        o_ref[...] = jnp.zeros_like(o_ref)

    o_ref[:, 0:1] = o_ref[:, 0:1] + ps
    o_ref[:, 1:2] = o_ref[:, 1:2] + pss


def _tc_reduce(ot2):
    return pl.pallas_call(
        _red_body,
        grid=(N_ACC // RED_TILE,),
        in_specs=[pl.BlockSpec((C, RED_TILE), lambda i: (0, i))],
        out_specs=pl.BlockSpec((C, 128), lambda i: (0, 0)),
        out_shape=jax.ShapeDtypeStruct((C, 128), jnp.float32),
    )(ot2)


def _apply_body(x_ref, st_ref, gb_ref, o_ref):
    x = x_ref[...]
    inv_n = 1.0 / N
    mean = st_ref[:, 0:1] * inv_n
    var = st_ref[:, 1:2] * inv_n - mean * mean
    scale = gb_ref[:, 0:1] * lax.rsqrt(var + BN_EPS)
    shift = gb_ref[:, 1:2] - mean * scale
    y = jnp.maximum(x * scale + shift, 0.0)
    r = lax.broadcasted_iota(jnp.int32, (C, C), 0)
    cc = lax.broadcasted_iota(jnp.int32, (C, C), 1)
    eye = jnp.where(r == cc, 1.0, 0.0).astype(jnp.float32)
    o_ref[...] = lax.dot_general(
        y, eye, dimension_numbers=(((0,), (0,)), ((), ())),
        preferred_element_type=jnp.float32)


def _tc_apply(ot2, stats, gb):
    return pl.pallas_call(
        _apply_body,
        grid=(N_ACC // RED_TILE,),
        in_specs=[
            pl.BlockSpec((C, RED_TILE), lambda i: (0, i)),
            pl.BlockSpec((C, 128), lambda i: (0, 0)),
            pl.BlockSpec((C, 128), lambda i: (0, 0)),
        ],
        out_specs=pl.BlockSpec((RED_TILE, C), lambda i: (i, 0)),
        out_shape=jax.ShapeDtypeStruct((N, C), jnp.float32),
    )(ot2, stats, gb)


@jax.jit
def kernel(feats, edge_src, edge_dst, W, gamma, beta):
    src_p, dst_p = _tc_pad(edge_src.astype(jnp.int32),
                           edge_dst.astype(jnp.int32))
    src3 = src_p.reshape(NW, EW // 128, 128)
    dst2 = dst_p.reshape(IDX_ROWS, 128)

    g = _sc_gather(feats, src3)
    mt = _tc_matmul_t(g, W[:, :, 0::2], W[:, :, 1::2])
    ot = _sc_scatter(mt.reshape(C // 2, IDX_ROWS, 128), dst2)
    ot2 = ot.reshape(C, N_ACC)
    stats = _tc_reduce(ot2)
    gb = jnp.zeros((C, 128), jnp.float32)
    gb = gb.at[:, 0].set(gamma).at[:, 1].set(beta)
    return _tc_apply(ot2, stats, gb)
